# Initial kernel scaffold; baseline (speedup 1.0000x reference)
import functools

import jax
import jax.numpy as jnp
from jax import lax
from jax.experimental import pallas as pl
from jax.experimental.pallas import tpu as pltpu
from jax.experimental.pallas import tpu_sc as plsc

ATOM_DIM = 16
N_NODES = 10000
N_EDGES = 320000

GATHER_WIN = 128
EDGE_BLOCK = 4000
NUM_SUBCORES = 16
ROWS_PER_SUBCORE = N_NODES // NUM_SUBCORES


def _vector_mesh():
    return plsc.VectorSubcoreMesh(core_axis_name="core", subcore_axis_name="subcore")


def _gather_rows(table, idx_2d):
    n = idx_2d.shape[1]

    @functools.partial(
        pl.kernel,
        out_type=jax.ShapeDtypeStruct((n, ATOM_DIM), jnp.float32),
        mesh=_vector_mesh(),
    )
    def gather_kernel(table_hbm, idx_hbm, out_hbm):
        def body(i_vmem, o_vmem):
            pltpu.sync_copy(table_hbm.at[i_vmem.at[0]], o_vmem)

        pltpu.emit_pipeline(
            body,
            grid=(n // GATHER_WIN,),
            in_specs=[pl.BlockSpec((1, GATHER_WIN), lambda i: (0, i))],
            out_specs=[pl.BlockSpec((GATHER_WIN, ATOM_DIM), lambda i: (i, 0))],
            core_axis_name=("core", "subcore"),
            dimension_semantics=(pltpu.PARALLEL,),
        )(idx_hbm, out_hbm)

    return gather_kernel(table, idx_2d)


def _edge_transform_body(bond_ref, src_ref, w_ref, b_ref, out_ref):
    bf = jnp.dot(bond_ref[...], w_ref[...], preferred_element_type=jnp.float32)
    bf = bf + b_ref[...]
    src = src_ref[...]
    src_t = jnp.concatenate([src] * ATOM_DIM, axis=1)
    prod = bf * src_t
    r = lax.broadcasted_iota(jnp.int32, (ATOM_DIM * ATOM_DIM, ATOM_DIM), 0)
    c = lax.broadcasted_iota(jnp.int32, (ATOM_DIM * ATOM_DIM, ATOM_DIM), 1)
    s = (r // ATOM_DIM == c).astype(jnp.float32)
    out_ref[...] = jnp.dot(prod, s, preferred_element_type=jnp.float32)


def _edge_transform(bond_features, src_rows, w, bias_2d):
    e = bond_features.shape[0]
    d2 = ATOM_DIM * ATOM_DIM
    return pl.pallas_call(
        _edge_transform_body,
        grid=(e // EDGE_BLOCK,),
        in_specs=[
            pl.BlockSpec((EDGE_BLOCK, ATOM_DIM), lambda i: (i, 0)),
            pl.BlockSpec((EDGE_BLOCK, ATOM_DIM), lambda i: (i, 0)),
            pl.BlockSpec((ATOM_DIM, d2), lambda i: (0, 0)),
            pl.BlockSpec((1, d2), lambda i: (0, 0)),
        ],
        out_specs=pl.BlockSpec((EDGE_BLOCK, ATOM_DIM), lambda i: (i, 0)),
        out_shape=jax.ShapeDtypeStruct((e, ATOM_DIM), jnp.float32),
    )(bond_features, src_rows, w, bias_2d)


def _scatter_add(values, idx_2d):
    n = idx_2d.shape[1]

    @functools.partial(
        pl.kernel,
        out_type=jax.ShapeDtypeStruct((2, N_NODES, ATOM_DIM), jnp.float32),
        mesh=_vector_mesh(),
        scratch_types=[
            pltpu.VMEM_SHARED((N_NODES, ATOM_DIM), jnp.float32),
            pltpu.VMEM((ROWS_PER_SUBCORE, ATOM_DIM), jnp.float32),
        ],
    )
    def scatter_kernel(val_hbm, idx_hbm, out_hbm, acc_shared, zero_vmem):
        core = lax.axis_index("core")
        sub = lax.axis_index("subcore")

        @pl.loop(0, ROWS_PER_SUBCORE)
        def _(i):
            zero_vmem[i, :] = jnp.zeros((ATOM_DIM,), jnp.float32)

        row0 = sub * ROWS_PER_SUBCORE
        pltpu.sync_copy(
            zero_vmem, acc_shared.at[pl.ds(row0, ROWS_PER_SUBCORE), :]
        )
        plsc.subcore_barrier()

        def body(v_vmem, i_vmem):
            pltpu.sync_copy(v_vmem, acc_shared.at[i_vmem.at[0]], add=True)

        pltpu.emit_pipeline(
            body,
            grid=(n // GATHER_WIN,),
            in_specs=[
                pl.BlockSpec((GATHER_WIN, ATOM_DIM), lambda i: (i, 0)),
                pl.BlockSpec((1, GATHER_WIN), lambda i: (0, i)),
            ],
            core_axis_name=("core", "subcore"),
            dimension_semantics=(pltpu.PARALLEL,),
        )(val_hbm, idx_hbm)

        plsc.subcore_barrier()
        pltpu.sync_copy(
            acc_shared.at[pl.ds(row0, ROWS_PER_SUBCORE), :],
            out_hbm.at[core, pl.ds(row0, ROWS_PER_SUBCORE), :],
        )

    return scatter_kernel(values, idx_2d)


def _sum_partials_body(p_ref, o_ref):
    o_ref[...] = p_ref[0] + p_ref[1]


def _sum_partials(partials):
    return pl.pallas_call(
        _sum_partials_body,
        out_shape=jax.ShapeDtypeStruct((N_NODES, ATOM_DIM), jnp.float32),
    )(partials)


def kernel(atom_features, bond_features, pair_indices, kernel, bias):
    idx_src = pair_indices[:, 1].astype(jnp.int32).reshape(1, N_EDGES)
    idx_dst = pair_indices[:, 0].astype(jnp.int32).reshape(1, N_EDGES)
    bias_2d = bias.reshape(1, ATOM_DIM * ATOM_DIM)

    src_rows = _gather_rows(atom_features, idx_src)
    transformed = _edge_transform(bond_features, src_rows, kernel, bias_2d)
    partials = _scatter_add(transformed, idx_dst)
    return _sum_partials(partials)


# speedup vs baseline: 3.7998x; 3.7998x over previous
import functools

import jax
import jax.numpy as jnp
from jax import lax
from jax.experimental import pallas as pl
from jax.experimental.pallas import tpu as pltpu
from jax.experimental.pallas import tpu_sc as plsc

ATOM_DIM = 16
N_NODES = 10000
N_EDGES = 320000

GATHER_WIN = 128
EDGE_BLOCK = 4000
NUM_SUBCORES = 16
ROWS_PER_SUBCORE = N_NODES // NUM_SUBCORES


def _vector_mesh():
    return plsc.VectorSubcoreMesh(core_axis_name="core", subcore_axis_name="subcore")


def _gather_rows(table, idx_2d):
    n = idx_2d.shape[1]

    @functools.partial(
        pl.kernel,
        out_type=jax.ShapeDtypeStruct((n, ATOM_DIM), jnp.float32),
        mesh=_vector_mesh(),
        compiler_params=pltpu.CompilerParams(use_tc_tiling_on_sc=False),
    )
    def gather_kernel(table_hbm, idx_hbm, out_hbm):
        def body(i_vmem, o_vmem):
            pltpu.sync_copy(table_hbm.at[i_vmem.at[0]], o_vmem)

        pltpu.emit_pipeline(
            body,
            grid=(n // GATHER_WIN,),
            in_specs=[pl.BlockSpec((1, GATHER_WIN), lambda i: (0, i))],
            out_specs=[pl.BlockSpec((GATHER_WIN, ATOM_DIM), lambda i: (i, 0))],
            core_axis_name=("core", "subcore"),
            dimension_semantics=(pltpu.PARALLEL,),
        )(idx_hbm, out_hbm)

    return gather_kernel(table, idx_2d)


def _edge_transform_body(bond_ref, src_ref, w_ref, b_ref, out_ref):
    bf = jnp.dot(bond_ref[...], w_ref[...], preferred_element_type=jnp.float32)
    bf = bf + b_ref[...]
    src = src_ref[...]
    src_t = jnp.concatenate([src] * ATOM_DIM, axis=1)
    prod = bf * src_t
    r = lax.broadcasted_iota(jnp.int32, (ATOM_DIM * ATOM_DIM, ATOM_DIM), 0)
    c = lax.broadcasted_iota(jnp.int32, (ATOM_DIM * ATOM_DIM, ATOM_DIM), 1)
    s = (r // ATOM_DIM == c).astype(jnp.float32)
    out_ref[...] = jnp.dot(prod, s, preferred_element_type=jnp.float32)


def _edge_transform(bond_features, src_rows, w, bias_2d):
    e = bond_features.shape[0]
    d2 = ATOM_DIM * ATOM_DIM
    return pl.pallas_call(
        _edge_transform_body,
        grid=(e // EDGE_BLOCK,),
        in_specs=[
            pl.BlockSpec((EDGE_BLOCK, ATOM_DIM), lambda i: (i, 0)),
            pl.BlockSpec((EDGE_BLOCK, ATOM_DIM), lambda i: (i, 0)),
            pl.BlockSpec((ATOM_DIM, d2), lambda i: (0, 0)),
            pl.BlockSpec((1, d2), lambda i: (0, 0)),
        ],
        out_specs=pl.BlockSpec((EDGE_BLOCK, ATOM_DIM), lambda i: (i, 0)),
        out_shape=jax.ShapeDtypeStruct((e, ATOM_DIM), jnp.float32),
    )(bond_features, src_rows, w, bias_2d)


def _scatter_add(values, idx_2d):
    n = idx_2d.shape[1]

    @functools.partial(
        pl.kernel,
        out_type=jax.ShapeDtypeStruct((2, N_NODES, ATOM_DIM), jnp.float32),
        mesh=_vector_mesh(),
        compiler_params=pltpu.CompilerParams(use_tc_tiling_on_sc=False),
        scratch_types=[
            pltpu.VMEM_SHARED((N_NODES, ATOM_DIM), jnp.float32),
            pltpu.VMEM((ROWS_PER_SUBCORE, ATOM_DIM), jnp.float32),
        ],
    )
    def scatter_kernel(val_hbm, idx_hbm, out_hbm, acc_shared, zero_vmem):
        core = lax.axis_index("core")
        sub = lax.axis_index("subcore")

        @pl.loop(0, ROWS_PER_SUBCORE)
        def _(i):
            zero_vmem[i, :] = jnp.zeros((ATOM_DIM,), jnp.float32)

        row0 = sub * ROWS_PER_SUBCORE
        pltpu.sync_copy(
            zero_vmem, acc_shared.at[pl.ds(row0, ROWS_PER_SUBCORE), :]
        )
        plsc.subcore_barrier()

        def body(v_vmem, i_vmem):
            pltpu.sync_copy(v_vmem, acc_shared.at[i_vmem.at[0]], add=True)

        pltpu.emit_pipeline(
            body,
            grid=(n // GATHER_WIN,),
            in_specs=[
                pl.BlockSpec((GATHER_WIN, ATOM_DIM), lambda i: (i, 0)),
                pl.BlockSpec((1, GATHER_WIN), lambda i: (0, i)),
            ],
            core_axis_name=("core", "subcore"),
            dimension_semantics=(pltpu.PARALLEL,),
        )(val_hbm, idx_hbm)

        plsc.subcore_barrier()
        pltpu.sync_copy(
            acc_shared.at[pl.ds(row0, ROWS_PER_SUBCORE), :],
            out_hbm.at[core, pl.ds(row0, ROWS_PER_SUBCORE), :],
        )

    return scatter_kernel(values, idx_2d)


def _sum_partials_body(p_ref, o_ref):
    o_ref[...] = p_ref[0] + p_ref[1]


def _sum_partials(partials):
    return pl.pallas_call(
        _sum_partials_body,
        out_shape=jax.ShapeDtypeStruct((N_NODES, ATOM_DIM), jnp.float32),
    )(partials)


def kernel(atom_features, bond_features, pair_indices, kernel, bias):
    idx_src = pair_indices[:, 1].astype(jnp.int32).reshape(1, N_EDGES)
    idx_dst = pair_indices[:, 0].astype(jnp.int32).reshape(1, N_EDGES)
    bias_2d = bias.reshape(1, ATOM_DIM * ATOM_DIM)

    src_rows = _gather_rows(atom_features, idx_src)
    transformed = _edge_transform(bond_features, src_rows, kernel, bias_2d)
    partials = _scatter_add(transformed, idx_dst)
    return _sum_partials(partials)


# speedup vs baseline: 4.5447x; 1.1960x over previous
import functools

import jax
import jax.numpy as jnp
from jax import lax
from jax.experimental import pallas as pl
from jax.experimental.pallas import tpu as pltpu
from jax.experimental.pallas import tpu_sc as plsc

ATOM_DIM = 16
N_NODES = 10000
N_EDGES = 320000

GATHER_WIN = 128
EDGE_BLOCK = 4000
NUM_SUBCORES = 16
ROWS_PER_SUBCORE = N_NODES // NUM_SUBCORES


def _vector_mesh():
    return plsc.VectorSubcoreMesh(core_axis_name="core", subcore_axis_name="subcore")


def _gather_rows(table, idx_2d):
    n = idx_2d.shape[1]

    @functools.partial(
        pl.kernel,
        out_type=jax.ShapeDtypeStruct((n, ATOM_DIM), jnp.float32),
        mesh=_vector_mesh(),
        compiler_params=pltpu.CompilerParams(use_tc_tiling_on_sc=False),
    )
    def gather_kernel(table_hbm, idx_hbm, out_hbm):
        def body(i_vmem, o_vmem):
            pltpu.sync_copy(table_hbm.at[i_vmem.at[0]], o_vmem)

        pltpu.emit_pipeline(
            body,
            grid=(n // GATHER_WIN,),
            in_specs=[pl.BlockSpec((1, GATHER_WIN), lambda i: (0, i))],
            out_specs=[pl.BlockSpec((GATHER_WIN, ATOM_DIM), lambda i: (i, 0))],
            core_axis_name=("core", "subcore"),
            dimension_semantics=(pltpu.PARALLEL,),
        )(idx_hbm, out_hbm)

    return gather_kernel(table, idx_2d)


def _edge_transform_body(bond_ref, src_ref, w_ref, t_ref, s_ref, b_ref, out_ref):
    bond = bond_ref[...]
    src = src_ref[...]
    bf = jnp.dot(bond, w_ref[...], preferred_element_type=jnp.float32) + b_ref[...]
    src_t = jnp.dot(src, t_ref[...], preferred_element_type=jnp.float32)
    out_ref[...] = jnp.dot(bf * src_t, s_ref[...], preferred_element_type=jnp.float32)


def _edge_transform(bond_features, src_rows, w, tmat, smat, bias_2d):
    e = bond_features.shape[0]
    d2 = ATOM_DIM * ATOM_DIM
    return pl.pallas_call(
        _edge_transform_body,
        grid=(e // EDGE_BLOCK,),
        in_specs=[
            pl.BlockSpec((EDGE_BLOCK, ATOM_DIM), lambda i: (i, 0)),
            pl.BlockSpec((EDGE_BLOCK, ATOM_DIM), lambda i: (i, 0)),
            pl.BlockSpec((ATOM_DIM, d2), lambda i: (0, 0)),
            pl.BlockSpec((ATOM_DIM, d2), lambda i: (0, 0)),
            pl.BlockSpec((d2, ATOM_DIM), lambda i: (0, 0)),
            pl.BlockSpec((1, d2), lambda i: (0, 0)),
        ],
        out_specs=pl.BlockSpec((EDGE_BLOCK, ATOM_DIM), lambda i: (i, 0)),
        out_shape=jax.ShapeDtypeStruct((e, ATOM_DIM), jnp.float32),
    )(bond_features, src_rows, w, tmat, smat, bias_2d)


def _scatter_add(values, idx_2d):
    n = idx_2d.shape[1]

    @functools.partial(
        pl.kernel,
        out_type=jax.ShapeDtypeStruct((2, N_NODES, ATOM_DIM), jnp.float32),
        mesh=_vector_mesh(),
        compiler_params=pltpu.CompilerParams(use_tc_tiling_on_sc=False),
        scratch_types=[
            pltpu.VMEM_SHARED((N_NODES, ATOM_DIM), jnp.float32),
            pltpu.VMEM((ROWS_PER_SUBCORE, ATOM_DIM), jnp.float32),
        ],
    )
    def scatter_kernel(val_hbm, idx_hbm, out_hbm, acc_shared, zero_vmem):
        core = lax.axis_index("core")
        sub = lax.axis_index("subcore")

        @pl.loop(0, ROWS_PER_SUBCORE)
        def _(i):
            zero_vmem[i, :] = jnp.zeros((ATOM_DIM,), jnp.float32)

        row0 = sub * ROWS_PER_SUBCORE
        pltpu.sync_copy(
            zero_vmem, acc_shared.at[pl.ds(row0, ROWS_PER_SUBCORE), :]
        )
        plsc.subcore_barrier()

        def body(v_vmem, i_vmem):
            pltpu.sync_copy(v_vmem, acc_shared.at[i_vmem.at[0]], add=True)

        pltpu.emit_pipeline(
            body,
            grid=(n // GATHER_WIN,),
            in_specs=[
                pl.BlockSpec((GATHER_WIN, ATOM_DIM), lambda i: (i, 0)),
                pl.BlockSpec((1, GATHER_WIN), lambda i: (0, i)),
            ],
            core_axis_name=("core", "subcore"),
            dimension_semantics=(pltpu.PARALLEL,),
        )(val_hbm, idx_hbm)

        plsc.subcore_barrier()
        pltpu.sync_copy(
            acc_shared.at[pl.ds(row0, ROWS_PER_SUBCORE), :],
            out_hbm.at[core, pl.ds(row0, ROWS_PER_SUBCORE), :],
        )

    return scatter_kernel(values, idx_2d)


def _sum_partials_body(p_ref, o_ref):
    o_ref[...] = p_ref[0] + p_ref[1]


def _sum_partials(partials):
    return pl.pallas_call(
        _sum_partials_body,
        out_shape=jax.ShapeDtypeStruct((N_NODES, ATOM_DIM), jnp.float32),
    )(partials)


def kernel(atom_features, bond_features, pair_indices, kernel, bias):
    idx_src = pair_indices[:, 1].astype(jnp.int32).reshape(1, N_EDGES)
    idx_dst = pair_indices[:, 0].astype(jnp.int32).reshape(1, N_EDGES)
    d = ATOM_DIM
    m = jnp.arange(d * d)
    tmat = (m[None, :] % d == jnp.arange(d)[:, None]).astype(jnp.float32)
    smat = (m[:, None] // d == jnp.arange(d)[None, :]).astype(jnp.float32)
    bias_2d = bias.reshape(1, d * d)

    src_rows = _gather_rows(atom_features, idx_src)
    transformed = _edge_transform(bond_features, src_rows, kernel, tmat, smat, bias_2d)
    partials = _scatter_add(transformed, idx_dst)
    return _sum_partials(partials)


# speedup vs baseline: 6.9740x; 1.5345x over previous
import functools

import jax
import jax.numpy as jnp
from jax import lax
from jax.experimental import pallas as pl
from jax.experimental.pallas import tpu as pltpu
from jax.experimental.pallas import tpu_sc as plsc

ATOM_DIM = 16
N_NODES = 10000
N_EDGES = 320000

GATHER_WIN = 128
EDGE_BLOCK = 8000
NUM_SUBCORES = 16
ROWS_PER_SUBCORE = N_NODES // NUM_SUBCORES


def _vector_mesh():
    return plsc.VectorSubcoreMesh(core_axis_name="core", subcore_axis_name="subcore")


def _gather_rows(table, idx_2d):
    n = idx_2d.shape[1]

    @functools.partial(
        pl.kernel,
        out_type=jax.ShapeDtypeStruct((n, ATOM_DIM), jnp.float32),
        mesh=_vector_mesh(),
        compiler_params=pltpu.CompilerParams(use_tc_tiling_on_sc=False),
    )
    def gather_kernel(table_hbm, idx_hbm, out_hbm):
        def body(i_vmem, o_vmem):
            pltpu.sync_copy(table_hbm.at[i_vmem.at[0]], o_vmem)

        pltpu.emit_pipeline(
            body,
            grid=(n // GATHER_WIN,),
            in_specs=[pl.BlockSpec((1, GATHER_WIN), lambda i: (0, i))],
            out_specs=[pl.BlockSpec((GATHER_WIN, ATOM_DIM), lambda i: (i, 0))],
            core_axis_name=("core", "subcore"),
            dimension_semantics=(pltpu.PARALLEL,),
        )(idx_hbm, out_hbm)

    return gather_kernel(table, idx_2d)


PACK = 8
PACKED_BLOCK = EDGE_BLOCK // PACK


def _edge_transform_body(bond_ref, src_ref, w_ref, t_ref, s_ref, b_ref, out_ref):
    bond = bond_ref[...]
    src = src_ref[...]
    bf = jnp.dot(bond, w_ref[...], preferred_element_type=jnp.float32) + b_ref[...]
    src_t = jnp.dot(src, t_ref[...], preferred_element_type=jnp.float32)
    out_ref[...] = jnp.dot(bf * src_t, s_ref[...], preferred_element_type=jnp.float32)


def _edge_transform(bond_p, src_p, w_big, t_big, s_big, bias_big):
    ep = bond_p.shape[0]
    dp = PACK * ATOM_DIM
    d2p = PACK * ATOM_DIM * ATOM_DIM
    return pl.pallas_call(
        _edge_transform_body,
        grid=(ep // PACKED_BLOCK,),
        in_specs=[
            pl.BlockSpec((PACKED_BLOCK, dp), lambda i: (i, 0)),
            pl.BlockSpec((PACKED_BLOCK, dp), lambda i: (i, 0)),
            pl.BlockSpec((dp, d2p), lambda i: (0, 0)),
            pl.BlockSpec((dp, d2p), lambda i: (0, 0)),
            pl.BlockSpec((d2p, dp), lambda i: (0, 0)),
            pl.BlockSpec((1, d2p), lambda i: (0, 0)),
        ],
        out_specs=pl.BlockSpec((PACKED_BLOCK, dp), lambda i: (i, 0)),
        out_shape=jax.ShapeDtypeStruct((ep, dp), jnp.float32),
    )(bond_p, src_p, w_big, t_big, s_big, bias_big)


def _scatter_add(values, idx_2d):
    n = idx_2d.shape[1]

    @functools.partial(
        pl.kernel,
        out_type=jax.ShapeDtypeStruct((2, N_NODES, ATOM_DIM), jnp.float32),
        mesh=_vector_mesh(),
        compiler_params=pltpu.CompilerParams(use_tc_tiling_on_sc=False),
        scratch_types=[
            pltpu.VMEM_SHARED((N_NODES, ATOM_DIM), jnp.float32),
            pltpu.VMEM((ROWS_PER_SUBCORE, ATOM_DIM), jnp.float32),
        ],
    )
    def scatter_kernel(val_hbm, idx_hbm, out_hbm, acc_shared, zero_vmem):
        core = lax.axis_index("core")
        sub = lax.axis_index("subcore")

        @pl.loop(0, ROWS_PER_SUBCORE)
        def _(i):
            zero_vmem[i, :] = jnp.zeros((ATOM_DIM,), jnp.float32)

        row0 = sub * ROWS_PER_SUBCORE
        pltpu.sync_copy(
            zero_vmem, acc_shared.at[pl.ds(row0, ROWS_PER_SUBCORE), :]
        )
        plsc.subcore_barrier()

        def body(v_vmem, i_vmem):
            pltpu.sync_copy(v_vmem, acc_shared.at[i_vmem.at[0]], add=True)

        pltpu.emit_pipeline(
            body,
            grid=(n // GATHER_WIN,),
            in_specs=[
                pl.BlockSpec((GATHER_WIN, ATOM_DIM), lambda i: (i, 0)),
                pl.BlockSpec((1, GATHER_WIN), lambda i: (0, i)),
            ],
            core_axis_name=("core", "subcore"),
            dimension_semantics=(pltpu.PARALLEL,),
        )(val_hbm, idx_hbm)

        plsc.subcore_barrier()
        pltpu.sync_copy(
            acc_shared.at[pl.ds(row0, ROWS_PER_SUBCORE), :],
            out_hbm.at[core, pl.ds(row0, ROWS_PER_SUBCORE), :],
        )

    return scatter_kernel(values, idx_2d)


def _sum_partials_body(p_ref, o_ref):
    o_ref[...] = p_ref[0] + p_ref[1]


def _sum_partials(partials):
    return pl.pallas_call(
        _sum_partials_body,
        out_shape=jax.ShapeDtypeStruct((N_NODES, ATOM_DIM), jnp.float32),
    )(partials)


def kernel(atom_features, bond_features, pair_indices, kernel, bias):
    idx_src = pair_indices[:, 1].astype(jnp.int32).reshape(1, N_EDGES)
    idx_dst = pair_indices[:, 0].astype(jnp.int32).reshape(1, N_EDGES)
    d = ATOM_DIM
    m = jnp.arange(d * d)
    tmat = (m[None, :] % d == jnp.arange(d)[:, None]).astype(jnp.float32)
    smat = (m[:, None] // d == jnp.arange(d)[None, :]).astype(jnp.float32)
    eye8 = jnp.eye(PACK, dtype=jnp.float32)
    w_big = jnp.kron(eye8, kernel)
    t_big = jnp.kron(eye8, tmat)
    s_big = jnp.kron(eye8, smat)
    bias_big = jnp.tile(bias, PACK).reshape(1, PACK * d * d)

    src_rows = _gather_rows(atom_features, idx_src)
    bond_p = bond_features.reshape(N_EDGES // PACK, PACK * d)
    src_p = src_rows.reshape(N_EDGES // PACK, PACK * d)
    transformed_p = _edge_transform(bond_p, src_p, w_big, t_big, s_big, bias_big)
    transformed = transformed_p.reshape(N_EDGES, d)
    partials = _scatter_add(transformed, idx_dst)
    return _sum_partials(partials)


# speedup vs baseline: 7.2487x; 1.0394x over previous
import functools

import jax
import jax.numpy as jnp
from jax import lax
from jax.experimental import pallas as pl
from jax.experimental.pallas import tpu as pltpu
from jax.experimental.pallas import tpu_sc as plsc

ATOM_DIM = 16
N_NODES = 10000
N_EDGES = 320000

GATHER_WIN = 128
STREAM_BATCH = 4
EDGE_BLOCK = 8000
NUM_SUBCORES = 16
ROWS_PER_SUBCORE = N_NODES // NUM_SUBCORES


def _vector_mesh():
    return plsc.VectorSubcoreMesh(core_axis_name="core", subcore_axis_name="subcore")


def _gather_rows(table, idx_rows):
    n = idx_rows.shape[0] * GATHER_WIN

    @functools.partial(
        pl.kernel,
        out_type=jax.ShapeDtypeStruct((n, ATOM_DIM), jnp.float32),
        mesh=_vector_mesh(),
        compiler_params=pltpu.CompilerParams(use_tc_tiling_on_sc=False),
        scratch_types=[pltpu.SemaphoreType.DMA],
    )
    def gather_kernel(table_hbm, idx_hbm, out_hbm, sem):
        def body(i_vmem, o_vmem):
            cps = [
                pltpu.async_copy(
                    table_hbm.at[i_vmem.at[j]],
                    o_vmem.at[pl.ds(j * GATHER_WIN, GATHER_WIN), :],
                    sem,
                )
                for j in range(STREAM_BATCH)
            ]
            for cp in cps:
                cp.wait()

        pltpu.emit_pipeline(
            body,
            grid=(n // (GATHER_WIN * STREAM_BATCH),),
            in_specs=[pl.BlockSpec((STREAM_BATCH, GATHER_WIN), lambda i: (i, 0))],
            out_specs=[
                pl.BlockSpec((STREAM_BATCH * GATHER_WIN, ATOM_DIM), lambda i: (i, 0))
            ],
            core_axis_name=("core", "subcore"),
            dimension_semantics=(pltpu.PARALLEL,),
        )(idx_hbm, out_hbm)

    return gather_kernel(table, idx_rows)


PACK = 8
PACKED_BLOCK = EDGE_BLOCK // PACK


def _edge_transform_body(bond_ref, src_ref, w_ref, t_ref, s_ref, b_ref, out_ref):
    bond = bond_ref[...]
    src = src_ref[...]
    bf = jnp.dot(bond, w_ref[...], preferred_element_type=jnp.float32) + b_ref[...]
    src_t = jnp.dot(src, t_ref[...], preferred_element_type=jnp.float32)
    out_ref[...] = jnp.dot(bf * src_t, s_ref[...], preferred_element_type=jnp.float32)


def _edge_transform(bond_p, src_p, w_big, t_big, s_big, bias_big):
    ep = bond_p.shape[0]
    dp = PACK * ATOM_DIM
    d2p = PACK * ATOM_DIM * ATOM_DIM
    return pl.pallas_call(
        _edge_transform_body,
        grid=(ep // PACKED_BLOCK,),
        in_specs=[
            pl.BlockSpec((PACKED_BLOCK, dp), lambda i: (i, 0)),
            pl.BlockSpec((PACKED_BLOCK, dp), lambda i: (i, 0)),
            pl.BlockSpec((dp, d2p), lambda i: (0, 0)),
            pl.BlockSpec((dp, d2p), lambda i: (0, 0)),
            pl.BlockSpec((d2p, dp), lambda i: (0, 0)),
            pl.BlockSpec((1, d2p), lambda i: (0, 0)),
        ],
        out_specs=pl.BlockSpec((PACKED_BLOCK, dp), lambda i: (i, 0)),
        out_shape=jax.ShapeDtypeStruct((ep, dp), jnp.float32),
    )(bond_p, src_p, w_big, t_big, s_big, bias_big)


def _scatter_add(values, idx_rows):
    n = idx_rows.shape[0] * GATHER_WIN

    @functools.partial(
        pl.kernel,
        out_type=jax.ShapeDtypeStruct((2, N_NODES, ATOM_DIM), jnp.float32),
        mesh=_vector_mesh(),
        compiler_params=pltpu.CompilerParams(use_tc_tiling_on_sc=False),
        scratch_types=[
            pltpu.VMEM_SHARED((N_NODES, ATOM_DIM), jnp.float32),
            pltpu.VMEM((ROWS_PER_SUBCORE, ATOM_DIM), jnp.float32),
            pltpu.SemaphoreType.DMA,
        ],
    )
    def scatter_kernel(val_hbm, idx_hbm, out_hbm, acc_shared, zero_vmem, sem):
        core = lax.axis_index("core")
        sub = lax.axis_index("subcore")

        @pl.loop(0, ROWS_PER_SUBCORE)
        def _(i):
            zero_vmem[i, :] = jnp.zeros((ATOM_DIM,), jnp.float32)

        row0 = sub * ROWS_PER_SUBCORE
        pltpu.sync_copy(
            zero_vmem, acc_shared.at[pl.ds(row0, ROWS_PER_SUBCORE), :]
        )
        plsc.subcore_barrier()

        def body(v_vmem, i_vmem):
            cps = [
                pltpu.async_copy(
                    v_vmem.at[pl.ds(j * GATHER_WIN, GATHER_WIN), :],
                    acc_shared.at[i_vmem.at[j]],
                    sem,
                    add=True,
                )
                for j in range(STREAM_BATCH)
            ]
            for cp in cps:
                cp.wait()

        pltpu.emit_pipeline(
            body,
            grid=(n // (GATHER_WIN * STREAM_BATCH),),
            in_specs=[
                pl.BlockSpec((STREAM_BATCH * GATHER_WIN, ATOM_DIM), lambda i: (i, 0)),
                pl.BlockSpec((STREAM_BATCH, GATHER_WIN), lambda i: (i, 0)),
            ],
            core_axis_name=("core", "subcore"),
            dimension_semantics=(pltpu.PARALLEL,),
        )(val_hbm, idx_hbm)

        plsc.subcore_barrier()
        pltpu.sync_copy(
            acc_shared.at[pl.ds(row0, ROWS_PER_SUBCORE), :],
            out_hbm.at[core, pl.ds(row0, ROWS_PER_SUBCORE), :],
        )

    return scatter_kernel(values, idx_rows)


def _sum_partials_body(p_ref, o_ref):
    o_ref[...] = p_ref[0] + p_ref[1]


def _sum_partials(partials):
    return pl.pallas_call(
        _sum_partials_body,
        out_shape=jax.ShapeDtypeStruct((N_NODES, ATOM_DIM), jnp.float32),
    )(partials)


def kernel(atom_features, bond_features, pair_indices, kernel, bias):
    idx_src = pair_indices[:, 1].astype(jnp.int32).reshape(N_EDGES // GATHER_WIN, GATHER_WIN)
    idx_dst = pair_indices[:, 0].astype(jnp.int32).reshape(N_EDGES // GATHER_WIN, GATHER_WIN)
    d = ATOM_DIM
    m = jnp.arange(d * d)
    tmat = (m[None, :] % d == jnp.arange(d)[:, None]).astype(jnp.float32)
    smat = (m[:, None] // d == jnp.arange(d)[None, :]).astype(jnp.float32)
    eye8 = jnp.eye(PACK, dtype=jnp.float32)
    w_big = jnp.kron(eye8, kernel)
    t_big = jnp.kron(eye8, tmat)
    s_big = jnp.kron(eye8, smat)
    bias_big = jnp.tile(bias, PACK).reshape(1, PACK * d * d)

    src_rows = _gather_rows(atom_features, idx_src)
    bond_p = bond_features.reshape(N_EDGES // PACK, PACK * d)
    src_p = src_rows.reshape(N_EDGES // PACK, PACK * d)
    transformed_p = _edge_transform(bond_p, src_p, w_big, t_big, s_big, bias_big)
    transformed = transformed_p.reshape(N_EDGES, d)
    partials = _scatter_add(transformed, idx_dst)
    return _sum_partials(partials)
